# baseline (speedup 1.0000x reference)
import functools

import jax
import jax.numpy as jnp
from jax import lax
from jax.experimental import pallas as pl
from jax.experimental.pallas import tpu as pltpu
from jax.experimental.pallas import tpu_sc as plsc

N_ROWS = 320000
D = 128
NUM_GRAPHS = 512
LANES = 16
N_SUBCORES = 16
NC = 2
ACC = NUM_GRAPHS * LANES
G_PER_TILE = NUM_GRAPHS // N_SUBCORES

TC_SHARE = 212992
SC_SHARE = N_ROWS - TC_SHARE
NW = NC * N_SUBCORES
ROWS_TILE = SC_SHARE // NW
CROWS = 209
NCHUNK = ROWS_TILE // CROWS

_R = 64
_TC_VIEW = TC_SHARE // D


_TC_HALF_BLOCKS = _TC_VIEW // _R // 2


def _energy_body(x_ref, o_ref):
    x = x_ref[...]
    o_ref[...] = 0.5 * jnp.sum(x * x, axis=-1)


def _make_energy(block_off):
    return pl.pallas_call(
        _energy_body,
        grid=(_TC_HALF_BLOCKS,),
        in_specs=[pl.BlockSpec((_R, D, D),
                               lambda i: (i + block_off, 0, 0))],
        out_specs=pl.BlockSpec((_R, D), lambda i: (i, 0)),
        out_shape=jax.ShapeDtypeStruct((_TC_HALF_BLOCKS * _R, D),
                                       jnp.float32),
        compiler_params=pltpu.CompilerParams(
            dimension_semantics=("arbitrary",),
        ),
    )


_energy0 = _make_energy(0)
_energy1 = _make_energy(_TC_HALF_BLOCKS)

_mesh1 = plsc.VectorSubcoreMesh(
    core_axis_name="c", subcore_axis_name="s", num_cores=1
)
_mesh2 = plsc.VectorSubcoreMesh(
    core_axis_name="c", subcore_axis_name="s", num_cores=2
)


@functools.partial(
    pl.kernel,
    mesh=_mesh2,
    out_type=jax.ShapeDtypeStruct((NC, NUM_GRAPHS), jnp.float32),
    scratch_types=[
        pltpu.VMEM((CROWS * D,), jnp.float32),
        pltpu.VMEM((CROWS * D,), jnp.float32),
        pltpu.VMEM((ROWS_TILE + LANES,), jnp.int32),
        pltpu.VMEM((ACC,), jnp.float32),
        pltpu.VMEM((G_PER_TILE * LANES,), jnp.float32),
        pltpu.VMEM((N_SUBCORES, G_PER_TILE * LANES), jnp.float32),
        pltpu.VMEM((G_PER_TILE,), jnp.float32),
        pltpu.VMEM_SHARED((N_SUBCORES, ACC), jnp.float32),
        pltpu.SemaphoreType.DMA,
        pltpu.SemaphoreType.DMA,
    ],
    compiler_params=pltpu.CompilerParams(needs_layout_passes=False),
)
def _energy_scatter_sc(x_hbm, b_hbm, out_hbm, xb0, xb1, b_v, acc_v, sum_v,
                       stage_v, res_v, shared, sem0, sem1):
    cid = lax.axis_index("c")
    sid = lax.axis_index("s")
    wid = sid * NC + cid
    rbase = TC_SHARE + wid * ROWS_TILE

    pltpu.sync_copy(b_hbm.at[pl.ds(rbase, ROWS_TILE)],
                    b_v.at[pl.ds(0, ROWS_TILE)])

    zeros16 = jnp.zeros((LANES,), jnp.float32)

    @plsc.parallel_loop(0, ACC // LANES, unroll=8)
    def _zero(i):
        acc_v[pl.ds(i * LANES, LANES)] = zeros16

    lane = lax.iota(jnp.int32, LANES)
    bufs = (xb0, xb1)
    sems = (sem0, sem1)
    cps = [None, None]
    cps[0] = pltpu.async_copy(
        x_hbm.at[pl.ds(rbase * D, CROWS * D)], xb0, sem0
    )
    for ch in range(NCHUNK):
        cur = ch % 2
        if ch + 1 < NCHUNK:
            nxt = 1 - cur
            cps[nxt] = pltpu.async_copy(
                x_hbm.at[pl.ds((rbase + (ch + 1) * CROWS) * D, CROWS * D)],
                bufs[nxt], sems[nxt],
            )
        cps[cur].wait()
        xb = bufs[cur]
        boff = ch * CROWS

        @plsc.parallel_loop(0, CROWS, unroll=11)
        def _rows(r):
            bval = b_v[pl.ds(boff + r, LANES)][0]
            off = r * D
            sq = [None] * (D // LANES)
            for k in range(D // LANES):
                v = xb[pl.ds(off + k * LANES, LANES)]
                sq[k] = v * v
            n = D // LANES
            while n > 1:
                for k in range(n // 2):
                    sq[k] = sq[2 * k] + sq[2 * k + 1]
                n //= 2
            plsc.addupdate_scatter(acc_v, [bval * LANES + lane],
                                   sq[0] * 0.5)

    pltpu.sync_copy(acc_v, shared.at[sid])
    plsc.subcore_barrier()

    goff = sid * G_PER_TILE * LANES
    pltpu.sync_copy(shared.at[:, pl.ds(goff, G_PER_TILE * LANES)], stage_v)

    @plsc.parallel_loop(0, G_PER_TILE, unroll=4)
    def _fold_tiles(c):
        s = pl.ds(c * LANES, LANES)
        tot = stage_v[0, s]
        for t in range(1, N_SUBCORES):
            tot = tot + stage_v[t, s]
        sum_v[s] = tot

    for c in range(G_PER_TILE // LANES):
        addr = c * LANES * LANES + lane * LANES
        tot = plsc.load_gather(sum_v, [addr])
        for l in range(1, LANES):
            tot = tot + plsc.load_gather(sum_v, [addr + l])
        res_v[pl.ds(c * LANES, LANES)] = tot

    pltpu.sync_copy(
        res_v, out_hbm.at[cid, pl.ds(sid * G_PER_TILE, G_PER_TILE)]
    )


_E_HALF = TC_SHARE // 2
_CHUNK_B = _E_HALF // N_SUBCORES


def _make_scatter_b(koff, with_prev):
    scratch = [
        pltpu.VMEM((_CHUNK_B,), jnp.float32),
        pltpu.VMEM((_CHUNK_B,), jnp.int32),
        pltpu.VMEM((ACC,), jnp.float32),
        pltpu.VMEM((G_PER_TILE * LANES,), jnp.float32),
        pltpu.VMEM((N_SUBCORES, G_PER_TILE * LANES), jnp.float32),
        pltpu.VMEM((G_PER_TILE,), jnp.float32),
        pltpu.VMEM((G_PER_TILE,), jnp.float32),
        pltpu.VMEM((G_PER_TILE,), jnp.float32),
        pltpu.VMEM((G_PER_TILE,), jnp.float32),
        pltpu.VMEM_SHARED((N_SUBCORES, ACC), jnp.float32),
    ]

    @functools.partial(
        pl.kernel,
        mesh=_mesh1,
        out_type=jax.ShapeDtypeStruct((NUM_GRAPHS,), jnp.float32),
        scratch_types=scratch,
        compiler_params=pltpu.CompilerParams(needs_layout_passes=False),
    )
    def _scatter(e_hbm, b_hbm, *rest):
        if with_prev:
            (p0_hbm, pa_hbm, out_hbm, e_v, b_v, acc_v, sum_v, stage_v,
             res_v, p0_v, pa0_v, pa1_v, shared) = rest
        else:
            (out_hbm, e_v, b_v, acc_v, sum_v, stage_v,
             res_v, p0_v, pa0_v, pa1_v, shared) = rest
        sid = lax.axis_index("s")
        base = sid * _CHUNK_B
        gslice = pl.ds(sid * G_PER_TILE, G_PER_TILE)

        pltpu.sync_copy(e_hbm.at[pl.ds(base, _CHUNK_B)], e_v)
        pltpu.sync_copy(b_hbm.at[pl.ds(koff + base, _CHUNK_B)], b_v)
        if with_prev:
            pltpu.sync_copy(p0_hbm.at[gslice], p0_v)
            pltpu.sync_copy(pa_hbm.at[0, gslice], pa0_v)
            pltpu.sync_copy(pa_hbm.at[1, gslice], pa1_v)

        zeros16 = jnp.zeros((LANES,), jnp.float32)

        @plsc.parallel_loop(0, ACC // LANES, unroll=8)
        def _zero(i):
            acc_v[pl.ds(i * LANES, LANES)] = zeros16

        lane = lax.iota(jnp.int32, LANES)

        @plsc.parallel_loop(0, _CHUNK_B // LANES, unroll=8)
        def _accum(i):
            s = pl.ds(i * LANES, LANES)
            idx = b_v[s]
            ev = e_v[s]
            plsc.addupdate_scatter(acc_v, [idx * LANES + lane], ev)

        pltpu.sync_copy(acc_v, shared.at[sid])
        plsc.subcore_barrier()

        goff = sid * G_PER_TILE * LANES
        pltpu.sync_copy(shared.at[:, pl.ds(goff, G_PER_TILE * LANES)],
                        stage_v)

        @plsc.parallel_loop(0, G_PER_TILE, unroll=4)
        def _fold_tiles(c):
            s = pl.ds(c * LANES, LANES)
            tot = stage_v[0, s]
            for t in range(1, N_SUBCORES):
                tot = tot + stage_v[t, s]
            sum_v[s] = tot

        for c in range(G_PER_TILE // LANES):
            addr = c * LANES * LANES + lane * LANES
            tot = plsc.load_gather(sum_v, [addr])
            for l in range(1, LANES):
                tot = tot + plsc.load_gather(sum_v, [addr + l])
            s = pl.ds(c * LANES, LANES)
            if with_prev:
                tot = tot + p0_v[s] + pa0_v[s] + pa1_v[s]
            res_v[s] = tot

        pltpu.sync_copy(res_v, out_hbm.at[gslice])

    return _scatter


_scatter_b0 = _make_scatter_b(0, False)
_scatter_b1 = _make_scatter_b(_E_HALF, True)


def kernel(X, batch, num_graphs):
    del num_graphs
    b = batch.astype(jnp.int32)
    part_sc = _energy_scatter_sc(X.reshape(-1), b)
    Xv = X.reshape(2500, D, D)
    e0 = _energy0(Xv).reshape(-1)
    e1 = _energy1(Xv).reshape(-1)
    p0 = _scatter_b0(e0, b)
    return _scatter_b1(e1, b, p0, part_sc)


# speedup vs baseline: 1.0437x; 1.0437x over previous
import functools

import jax
import jax.numpy as jnp
from jax import lax
from jax.experimental import pallas as pl
from jax.experimental.pallas import tpu as pltpu
from jax.experimental.pallas import tpu_sc as plsc

N_ROWS = 320000
D = 128
NUM_GRAPHS = 512
LANES = 16
N_SUBCORES = 16
NC = 2
ACC = NUM_GRAPHS * LANES
G_PER_TILE = NUM_GRAPHS // N_SUBCORES

TC_SHARE = 204800
SC_SHARE = N_ROWS - TC_SHARE
NW = NC * N_SUBCORES
ROWS_TILE = SC_SHARE // NW
CROWS = 225
NCHUNK = ROWS_TILE // CROWS

_R = 64
_TC_VIEW = TC_SHARE // D


def _energy_body(x_ref, o_ref):
    x = x_ref[...]
    o_ref[...] = 0.5 * jnp.sum(x * x, axis=-1)


_energy_tc = pl.pallas_call(
    _energy_body,
    grid=(_TC_VIEW // _R,),
    in_specs=[pl.BlockSpec((_R, D, D), lambda i: (i, 0, 0))],
    out_specs=pl.BlockSpec((_R, D), lambda i: (i, 0)),
    out_shape=jax.ShapeDtypeStruct((_TC_VIEW, D), jnp.float32),
    compiler_params=pltpu.CompilerParams(
        dimension_semantics=("arbitrary",),
    ),
)

_mesh1 = plsc.VectorSubcoreMesh(
    core_axis_name="c", subcore_axis_name="s", num_cores=1
)
_mesh2 = plsc.VectorSubcoreMesh(
    core_axis_name="c", subcore_axis_name="s", num_cores=2
)


@functools.partial(
    pl.kernel,
    mesh=_mesh2,
    out_type=jax.ShapeDtypeStruct((NC, NUM_GRAPHS), jnp.float32),
    scratch_types=[
        pltpu.VMEM((CROWS * D,), jnp.float32),
        pltpu.VMEM((CROWS * D,), jnp.float32),
        pltpu.VMEM((ROWS_TILE + LANES,), jnp.int32),
        pltpu.VMEM((ACC,), jnp.float32),
        pltpu.VMEM((G_PER_TILE * LANES,), jnp.float32),
        pltpu.VMEM((N_SUBCORES, G_PER_TILE * LANES), jnp.float32),
        pltpu.VMEM((G_PER_TILE,), jnp.float32),
        pltpu.VMEM_SHARED((N_SUBCORES, ACC), jnp.float32),
        pltpu.SemaphoreType.DMA,
        pltpu.SemaphoreType.DMA,
    ],
    compiler_params=pltpu.CompilerParams(needs_layout_passes=False),
)
def _energy_scatter_sc(x_hbm, b_hbm, out_hbm, xb0, xb1, b_v, acc_v, sum_v,
                       stage_v, res_v, shared, sem0, sem1):
    cid = lax.axis_index("c")
    sid = lax.axis_index("s")
    wid = sid * NC + cid
    rbase = TC_SHARE + wid * ROWS_TILE

    pltpu.sync_copy(b_hbm.at[pl.ds(rbase, ROWS_TILE)],
                    b_v.at[pl.ds(0, ROWS_TILE)])

    zeros16 = jnp.zeros((LANES,), jnp.float32)

    @plsc.parallel_loop(0, ACC // LANES, unroll=8)
    def _zero(i):
        acc_v[pl.ds(i * LANES, LANES)] = zeros16

    lane = lax.iota(jnp.int32, LANES)
    bufs = (xb0, xb1)
    sems = (sem0, sem1)
    cps = [None, None]
    cps[0] = pltpu.async_copy(
        x_hbm.at[pl.ds(rbase * D, CROWS * D)], xb0, sem0
    )
    for ch in range(NCHUNK):
        cur = ch % 2
        if ch + 1 < NCHUNK:
            nxt = 1 - cur
            cps[nxt] = pltpu.async_copy(
                x_hbm.at[pl.ds((rbase + (ch + 1) * CROWS) * D, CROWS * D)],
                bufs[nxt], sems[nxt],
            )
        cps[cur].wait()
        xb = bufs[cur]
        boff = ch * CROWS

        @plsc.parallel_loop(0, CROWS, unroll=9)
        def _rows(r):
            bval = b_v[pl.ds(boff + r, LANES)][0]
            off = r * D
            sq = [None] * (D // LANES)
            for k in range(D // LANES):
                v = xb[pl.ds(off + k * LANES, LANES)]
                sq[k] = v * v
            n = D // LANES
            while n > 1:
                for k in range(n // 2):
                    sq[k] = sq[2 * k] + sq[2 * k + 1]
                n //= 2
            plsc.addupdate_scatter(acc_v, [bval * LANES + lane],
                                   sq[0] * 0.5)

    pltpu.sync_copy(acc_v, shared.at[sid])
    plsc.subcore_barrier()

    goff = sid * G_PER_TILE * LANES
    pltpu.sync_copy(shared.at[:, pl.ds(goff, G_PER_TILE * LANES)], stage_v)

    @plsc.parallel_loop(0, G_PER_TILE, unroll=4)
    def _fold_tiles(c):
        s = pl.ds(c * LANES, LANES)
        tot = stage_v[0, s]
        for t in range(1, N_SUBCORES):
            tot = tot + stage_v[t, s]
        sum_v[s] = tot

    for c in range(G_PER_TILE // LANES):
        addr = c * LANES * LANES + lane * LANES
        tot = plsc.load_gather(sum_v, [addr])
        for l in range(1, LANES):
            tot = tot + plsc.load_gather(sum_v, [addr + l])
        res_v[pl.ds(c * LANES, LANES)] = tot

    pltpu.sync_copy(
        res_v, out_hbm.at[cid, pl.ds(sid * G_PER_TILE, G_PER_TILE)]
    )


_CHUNK_B = TC_SHARE // N_SUBCORES


def _make_scatter_b(koff, with_prev):
    scratch = [
        pltpu.VMEM((_CHUNK_B,), jnp.float32),
        pltpu.VMEM((_CHUNK_B,), jnp.int32),
        pltpu.VMEM((ACC,), jnp.float32),
        pltpu.VMEM((G_PER_TILE * LANES,), jnp.float32),
        pltpu.VMEM((N_SUBCORES, G_PER_TILE * LANES), jnp.float32),
        pltpu.VMEM((G_PER_TILE,), jnp.float32),
        pltpu.VMEM((G_PER_TILE,), jnp.float32),
        pltpu.VMEM((G_PER_TILE,), jnp.float32),
        pltpu.VMEM((G_PER_TILE,), jnp.float32),
        pltpu.VMEM_SHARED((N_SUBCORES, ACC), jnp.float32),
    ]

    @functools.partial(
        pl.kernel,
        mesh=_mesh1,
        out_type=jax.ShapeDtypeStruct((NUM_GRAPHS,), jnp.float32),
        scratch_types=scratch,
        compiler_params=pltpu.CompilerParams(needs_layout_passes=False),
    )
    def _scatter(e_hbm, b_hbm, *rest):
        (pa_hbm, out_hbm, e_v, b_v, acc_v, sum_v, stage_v,
         res_v, p0_v, pa0_v, pa1_v, shared) = rest
        sid = lax.axis_index("s")
        base = sid * _CHUNK_B
        gslice = pl.ds(sid * G_PER_TILE, G_PER_TILE)

        pltpu.sync_copy(e_hbm.at[pl.ds(base, _CHUNK_B)], e_v)
        pltpu.sync_copy(b_hbm.at[pl.ds(koff + base, _CHUNK_B)], b_v)
        pltpu.sync_copy(pa_hbm.at[0, gslice], pa0_v)
        pltpu.sync_copy(pa_hbm.at[1, gslice], pa1_v)

        zeros16 = jnp.zeros((LANES,), jnp.float32)

        @plsc.parallel_loop(0, ACC // LANES, unroll=8)
        def _zero(i):
            acc_v[pl.ds(i * LANES, LANES)] = zeros16

        lane = lax.iota(jnp.int32, LANES)

        @plsc.parallel_loop(0, _CHUNK_B // LANES, unroll=8)
        def _accum(i):
            s = pl.ds(i * LANES, LANES)
            idx = b_v[s]
            ev = e_v[s]
            plsc.addupdate_scatter(acc_v, [idx * LANES + lane], ev)

        pltpu.sync_copy(acc_v, shared.at[sid])
        plsc.subcore_barrier()

        goff = sid * G_PER_TILE * LANES
        pltpu.sync_copy(shared.at[:, pl.ds(goff, G_PER_TILE * LANES)],
                        stage_v)

        @plsc.parallel_loop(0, G_PER_TILE, unroll=4)
        def _fold_tiles(c):
            s = pl.ds(c * LANES, LANES)
            tot = stage_v[0, s]
            for t in range(1, N_SUBCORES):
                tot = tot + stage_v[t, s]
            sum_v[s] = tot

        for c in range(G_PER_TILE // LANES):
            addr = c * LANES * LANES + lane * LANES
            tot = plsc.load_gather(sum_v, [addr])
            for l in range(1, LANES):
                tot = tot + plsc.load_gather(sum_v, [addr + l])
            s = pl.ds(c * LANES, LANES)
            res_v[s] = tot + pa0_v[s] + pa1_v[s]

        pltpu.sync_copy(res_v, out_hbm.at[gslice])

    return _scatter


_scatter_b = _make_scatter_b(0, True)


def kernel(X, batch, num_graphs):
    del num_graphs
    b = batch.astype(jnp.int32)
    part_sc = _energy_scatter_sc(X.reshape(-1), b)
    e = _energy_tc(X.reshape(2500, D, D)).reshape(-1)
    return _scatter_b(e, b, part_sc)


# speedup vs baseline: 1.0655x; 1.0208x over previous
import functools

import jax
import jax.numpy as jnp
from jax import lax
from jax.experimental import pallas as pl
from jax.experimental.pallas import tpu as pltpu
from jax.experimental.pallas import tpu_sc as plsc

N_ROWS = 320000
D = 128
NUM_GRAPHS = 512
LANES = 16
N_SUBCORES = 16
NC = 2
ACC = NUM_GRAPHS * LANES
G_PER_TILE = NUM_GRAPHS // N_SUBCORES

TC_SHARE = 204800
SC_SHARE = N_ROWS - TC_SHARE
NW = NC * N_SUBCORES
ROWS_TILE = SC_SHARE // NW
CROWS = 225
NCHUNK = ROWS_TILE // CROWS

_R = 64
_TC_VIEW = TC_SHARE // D


def _energy_body(x_ref, o_ref):
    x = x_ref[...]
    o_ref[...] = 0.5 * jnp.sum(x * x, axis=-1)


_energy_tc = pl.pallas_call(
    _energy_body,
    grid=(_TC_VIEW // _R,),
    in_specs=[pl.BlockSpec((_R, D, D), lambda i: (i, 0, 0))],
    out_specs=pl.BlockSpec((_R, D), lambda i: (i, 0)),
    out_shape=jax.ShapeDtypeStruct((_TC_VIEW, D), jnp.float32),
    compiler_params=pltpu.CompilerParams(
        dimension_semantics=("arbitrary",),
    ),
)

_mesh1 = plsc.VectorSubcoreMesh(
    core_axis_name="c", subcore_axis_name="s", num_cores=1
)
_mesh2 = plsc.VectorSubcoreMesh(
    core_axis_name="c", subcore_axis_name="s", num_cores=2
)


@functools.partial(
    pl.kernel,
    mesh=_mesh2,
    out_type=jax.ShapeDtypeStruct((NC, NUM_GRAPHS), jnp.float32),
    scratch_types=[
        pltpu.VMEM((CROWS * D,), jnp.float32),
        pltpu.VMEM((CROWS * D,), jnp.float32),
        pltpu.VMEM((ROWS_TILE + LANES,), jnp.int32),
        pltpu.VMEM((ACC,), jnp.float32),
        pltpu.VMEM((G_PER_TILE * LANES,), jnp.float32),
        pltpu.VMEM((N_SUBCORES, G_PER_TILE * LANES), jnp.float32),
        pltpu.VMEM((G_PER_TILE,), jnp.float32),
        pltpu.VMEM_SHARED((N_SUBCORES, ACC), jnp.float32),
        pltpu.SemaphoreType.DMA,
        pltpu.SemaphoreType.DMA,
    ],
    compiler_params=pltpu.CompilerParams(needs_layout_passes=False),
)
def _energy_scatter_sc(x_hbm, b_hbm, out_hbm, xb0, xb1, b_v, acc_v, sum_v,
                       stage_v, res_v, shared, sem0, sem1):
    cid = lax.axis_index("c")
    sid = lax.axis_index("s")
    wid = sid * NC + cid
    rbase = TC_SHARE + wid * ROWS_TILE

    pltpu.sync_copy(b_hbm.at[pl.ds(rbase, ROWS_TILE)],
                    b_v.at[pl.ds(0, ROWS_TILE)])

    zeros16 = jnp.zeros((LANES,), jnp.float32)

    @plsc.parallel_loop(0, ACC // LANES, unroll=8)
    def _zero(i):
        acc_v[pl.ds(i * LANES, LANES)] = zeros16

    lane = lax.iota(jnp.int32, LANES)
    bufs = (xb0, xb1)
    sems = (sem0, sem1)
    cps = [None, None]
    cps[0] = pltpu.async_copy(
        x_hbm.at[pl.ds(rbase * D, CROWS * D)], xb0, sem0
    )
    for ch in range(NCHUNK):
        cur = ch % 2
        if ch + 1 < NCHUNK:
            nxt = 1 - cur
            cps[nxt] = pltpu.async_copy(
                x_hbm.at[pl.ds((rbase + (ch + 1) * CROWS) * D, CROWS * D)],
                bufs[nxt], sems[nxt],
            )
        cps[cur].wait()
        xb = bufs[cur]
        boff = ch * CROWS

        @plsc.parallel_loop(0, CROWS, unroll=9)
        def _rows(r):
            bval = b_v[pl.ds(boff + r, LANES)][0]
            off = r * D
            sq = [None] * (D // LANES)
            for k in range(D // LANES):
                v = xb[pl.ds(off + k * LANES, LANES)]
                sq[k] = v * v
            n = D // LANES
            while n > 1:
                for k in range(n // 2):
                    sq[k] = sq[2 * k] + sq[2 * k + 1]
                n //= 2
            plsc.addupdate_scatter(acc_v, [bval * LANES + lane],
                                   sq[0] * 0.5)

    pltpu.sync_copy(acc_v, shared.at[sid])
    plsc.subcore_barrier()

    goff = sid * G_PER_TILE * LANES
    pltpu.sync_copy(shared.at[:, pl.ds(goff, G_PER_TILE * LANES)], stage_v)

    @plsc.parallel_loop(0, G_PER_TILE, unroll=4)
    def _fold_tiles(c):
        s = pl.ds(c * LANES, LANES)
        tot = stage_v[0, s]
        for t in range(1, N_SUBCORES):
            tot = tot + stage_v[t, s]
        sum_v[s] = tot

    for c in range(G_PER_TILE // LANES):
        addr = c * LANES * LANES + lane * LANES
        tot = plsc.load_gather(sum_v, [addr])
        for l in range(1, LANES):
            tot = tot + plsc.load_gather(sum_v, [addr + l])
        res_v[pl.ds(c * LANES, LANES)] = tot

    pltpu.sync_copy(
        res_v, out_hbm.at[cid, pl.ds(sid * G_PER_TILE, G_PER_TILE)]
    )


_CHUNK_B = TC_SHARE // N_SUBCORES


def _make_scatter_b(koff, with_prev):
    scratch = [
        pltpu.VMEM((_CHUNK_B,), jnp.float32),
        pltpu.VMEM((_CHUNK_B,), jnp.int32),
        pltpu.VMEM((ACC,), jnp.float32),
        pltpu.VMEM((G_PER_TILE * LANES,), jnp.float32),
        pltpu.VMEM((N_SUBCORES, G_PER_TILE * LANES), jnp.float32),
        pltpu.VMEM((G_PER_TILE,), jnp.float32),
        pltpu.VMEM((G_PER_TILE,), jnp.float32),
        pltpu.VMEM((G_PER_TILE,), jnp.float32),
        pltpu.VMEM((G_PER_TILE,), jnp.float32),
        pltpu.VMEM_SHARED((N_SUBCORES, ACC), jnp.float32),
        pltpu.SemaphoreType.DMA,
    ]

    @functools.partial(
        pl.kernel,
        mesh=_mesh1,
        out_type=jax.ShapeDtypeStruct((NUM_GRAPHS,), jnp.float32),
        scratch_types=scratch,
        compiler_params=pltpu.CompilerParams(needs_layout_passes=False),
    )
    def _scatter(e_hbm, b_hbm, *rest):
        (pa_hbm, out_hbm, e_v, b_v, acc_v, sum_v, stage_v,
         res_v, p0_v, pa0_v, pa1_v, shared, dsem) = rest
        sid = lax.axis_index("s")
        base = sid * _CHUNK_B
        gslice = pl.ds(sid * G_PER_TILE, G_PER_TILE)

        cp_e = pltpu.async_copy(e_hbm.at[pl.ds(base, _CHUNK_B)], e_v, dsem)
        cp_b = pltpu.async_copy(b_hbm.at[pl.ds(koff + base, _CHUNK_B)],
                                b_v, dsem)
        pltpu.sync_copy(pa_hbm.at[0, gslice], pa0_v)
        pltpu.sync_copy(pa_hbm.at[1, gslice], pa1_v)

        zeros16 = jnp.zeros((LANES,), jnp.float32)

        @plsc.parallel_loop(0, ACC // LANES, unroll=8)
        def _zero(i):
            acc_v[pl.ds(i * LANES, LANES)] = zeros16

        cp_e.wait()
        cp_b.wait()

        lane = lax.iota(jnp.int32, LANES)

        @plsc.parallel_loop(0, _CHUNK_B // LANES, unroll=8)
        def _accum(i):
            s = pl.ds(i * LANES, LANES)
            idx = b_v[s]
            ev = e_v[s]
            plsc.addupdate_scatter(acc_v, [idx * LANES + lane], ev)

        pltpu.sync_copy(acc_v, shared.at[sid])
        plsc.subcore_barrier()

        goff = sid * G_PER_TILE * LANES
        pltpu.sync_copy(shared.at[:, pl.ds(goff, G_PER_TILE * LANES)],
                        stage_v)

        @plsc.parallel_loop(0, G_PER_TILE, unroll=4)
        def _fold_tiles(c):
            s = pl.ds(c * LANES, LANES)
            tot = stage_v[0, s]
            for t in range(1, N_SUBCORES):
                tot = tot + stage_v[t, s]
            sum_v[s] = tot

        for c in range(G_PER_TILE // LANES):
            addr = c * LANES * LANES + lane * LANES
            tot = plsc.load_gather(sum_v, [addr])
            for l in range(1, LANES):
                tot = tot + plsc.load_gather(sum_v, [addr + l])
            s = pl.ds(c * LANES, LANES)
            res_v[s] = tot + pa0_v[s] + pa1_v[s]

        pltpu.sync_copy(res_v, out_hbm.at[gslice])

    return _scatter


_scatter_b = _make_scatter_b(0, True)


def kernel(X, batch, num_graphs):
    del num_graphs
    b = batch.astype(jnp.int32)
    part_sc = _energy_scatter_sc(X.reshape(-1), b)
    e = _energy_tc(X.reshape(2500, D, D)).reshape(-1)
    return _scatter_b(e, b, part_sc)
